# baseline (speedup 1.0000x reference)
import functools

import jax
import jax.numpy as jnp
from jax import lax
from jax.experimental import pallas as pl
from jax.experimental.pallas import tpu as pltpu
from jax.experimental.pallas import tpu_sc as plsc

BUFFER_SIZE = 1000000
OBS_DIM = 64
ACT_DIM = 16
BATCH = 4096

_NUM_CORES = 2
_NUM_SUBCORES = 16
_NW = _NUM_CORES * _NUM_SUBCORES
_BPW = BATCH // _NW

_MESH = dict(core_axis_name="c", subcore_axis_name="s")
_LINEAR = pltpu.CompilerParams(use_tc_tiling_on_sc=False)


def _row_gather_kernel(tab_hbm, idx_hbm, out, idx_v, buf, s0):
    wid = lax.axis_index("s") * _NUM_CORES + lax.axis_index("c")
    base = wid * _BPW
    dim = tab_hbm.shape[0]
    pltpu.sync_copy(idx_hbm.at[pl.ds(base, _BPW)], idx_v)
    copies = [
        pltpu.async_copy(tab_hbm.at[c].at[idx_v],
                         buf.at[pl.ds(c * _BPW, _BPW)], s0)
        for c in range(dim)
    ]
    for cp in copies:
        cp.wait()
    for c in range(dim):
        pltpu.sync_copy(buf.at[pl.ds(c * _BPW, _BPW)],
                        out.at[c, pl.ds(base, _BPW)])


def _make_row_gather(dim):
    return functools.partial(
        pl.kernel,
        mesh=plsc.VectorSubcoreMesh(**_MESH),
        compiler_params=_LINEAR,
        out_type=jax.ShapeDtypeStruct((dim, BATCH), jnp.float32),
        scratch_types=[
            pltpu.VMEM((_BPW,), jnp.int32),
            pltpu.VMEM((dim * _BPW,), jnp.float32),
            pltpu.SemaphoreType.DMA,
        ],
    )(_row_gather_kernel)


def _scalar_gather_kernel(rew_hbm, done_hbm, idx_hbm, out_rew, out_done,
                          idx_v, rew_v, done_v, s0, s1):
    wid = lax.axis_index("s") * _NUM_CORES + lax.axis_index("c")
    base = wid * _BPW
    pltpu.sync_copy(idx_hbm.at[pl.ds(base, _BPW)], idx_v)
    c_rew = pltpu.async_copy(rew_hbm.at[idx_v], rew_v, s0)
    c_done = pltpu.async_copy(done_hbm.at[idx_v], done_v, s1)
    c_rew.wait()
    pltpu.sync_copy(rew_v, out_rew.at[pl.ds(base, _BPW)])
    c_done.wait()
    pltpu.sync_copy(done_v, out_done.at[pl.ds(base, _BPW)])


_scalar_gather = functools.partial(
    pl.kernel,
    mesh=plsc.VectorSubcoreMesh(**_MESH),
    compiler_params=_LINEAR,
    out_type=[
        jax.ShapeDtypeStruct((BATCH,), jnp.float32),
        jax.ShapeDtypeStruct((BATCH,), jnp.float32),
    ],
    scratch_types=[
        pltpu.VMEM((_BPW,), jnp.int32),
        pltpu.VMEM((_BPW,), jnp.float32),
        pltpu.VMEM((_BPW,), jnp.float32),
        pltpu.SemaphoreType.DMA,
        pltpu.SemaphoreType.DMA,
    ],
)(_scalar_gather_kernel)


@jax.jit
def _sample(observations, actions, rewards, next_observations, dones, indices):
    obs_b = _make_row_gather(OBS_DIM)(observations.T, indices)
    nobs_b = _make_row_gather(OBS_DIM)(next_observations.T, indices)
    act_b = _make_row_gather(ACT_DIM)(actions.T, indices)
    rew_b, done_b = _scalar_gather(rewards, dones, indices)
    return obs_b.T, act_b.T, rew_b, nobs_b.T, done_b


def kernel(observations, actions, rewards, next_observations, dones, indices):
    idx = indices.astype(jnp.int32)
    return tuple(_sample(observations, actions, rewards, next_observations,
                         dones, idx))


# speedup vs baseline: 7.9069x; 7.9069x over previous
import functools

import jax
import jax.numpy as jnp
from jax import lax
from jax.experimental import pallas as pl
from jax.experimental.pallas import tpu as pltpu
from jax.experimental.pallas import tpu_sc as plsc

BUFFER_SIZE = 1000000
OBS_DIM = 64
ACT_DIM = 16
BATCH = 4096

_NUM_CORES = 2
_NUM_SUBCORES = 16
_NW = _NUM_CORES * _NUM_SUBCORES
_BPW = BATCH // _NW

_MESH = dict(core_axis_name="c", subcore_axis_name="s")
_LINEAR = pltpu.CompilerParams(use_tc_tiling_on_sc=False)


def _row_gather_kernel(tab_hbm, idx_hbm, out, idx_v, buf, s0):
    wid = lax.axis_index("s") * _NUM_CORES + lax.axis_index("c")
    base = wid * _BPW
    pltpu.sync_copy(idx_hbm.at[pl.ds(base, _BPW)], idx_v)
    pltpu.async_copy(tab_hbm.at[idx_v], buf, s0).wait()
    pltpu.sync_copy(buf, out.at[pl.ds(base, _BPW)])


def _make_row_gather(dim):
    return functools.partial(
        pl.kernel,
        mesh=plsc.VectorSubcoreMesh(**_MESH),
        compiler_params=_LINEAR,
        out_type=jax.ShapeDtypeStruct((BATCH, dim), jnp.float32),
        scratch_types=[
            pltpu.VMEM((_BPW,), jnp.int32),
            pltpu.VMEM((_BPW, dim), jnp.float32),
            pltpu.SemaphoreType.DMA,
        ],
    )(_row_gather_kernel)


def _scalar_gather_kernel(rew_hbm, done_hbm, idx_hbm, out_rew, out_done,
                          idx_v, rew_v, done_v, s0, s1):
    wid = lax.axis_index("s") * _NUM_CORES + lax.axis_index("c")
    base = wid * _BPW
    pltpu.sync_copy(idx_hbm.at[pl.ds(base, _BPW)], idx_v)
    c_rew = pltpu.async_copy(rew_hbm.at[idx_v], rew_v, s0)
    c_done = pltpu.async_copy(done_hbm.at[idx_v], done_v, s1)
    c_rew.wait()
    pltpu.sync_copy(rew_v, out_rew.at[pl.ds(base, _BPW)])
    c_done.wait()
    pltpu.sync_copy(done_v, out_done.at[pl.ds(base, _BPW)])


_scalar_gather = functools.partial(
    pl.kernel,
    mesh=plsc.VectorSubcoreMesh(**_MESH),
    compiler_params=_LINEAR,
    out_type=[
        jax.ShapeDtypeStruct((BATCH,), jnp.float32),
        jax.ShapeDtypeStruct((BATCH,), jnp.float32),
    ],
    scratch_types=[
        pltpu.VMEM((_BPW,), jnp.int32),
        pltpu.VMEM((_BPW,), jnp.float32),
        pltpu.VMEM((_BPW,), jnp.float32),
        pltpu.SemaphoreType.DMA,
        pltpu.SemaphoreType.DMA,
    ],
)(_scalar_gather_kernel)


@jax.jit
def _sample(observations, actions, rewards, next_observations, dones, indices):
    obs_b = _make_row_gather(OBS_DIM)(observations, indices)
    nobs_b = _make_row_gather(OBS_DIM)(next_observations, indices)
    act_b = _make_row_gather(ACT_DIM)(actions, indices)
    rew_b, done_b = _scalar_gather(rewards, dones, indices)
    return obs_b, act_b, rew_b, nobs_b, done_b


def kernel(observations, actions, rewards, next_observations, dones, indices):
    idx = indices.astype(jnp.int32)
    return tuple(_sample(observations, actions, rewards, next_observations,
                         dones, idx))
